# baseline (speedup 1.0000x reference)
import functools

import jax
import jax.numpy as jnp
import numpy as np
from jax import lax
from jax.experimental import pallas as pl
from jax.experimental.pallas import tpu as pltpu
from jax.experimental.pallas import tpu_sc as plsc

FR = 5
NPF = 32
TEMP = 10000.0
MM = 1_000_000
BB = 16384
NC, NS = 2, 16
NW = NC * NS
QW = BB // NW
WW = BB // NS
WR = WW // 128
QR = QW // 128
ROUNDS = 3
TAG_LEN = NC * MM + 128
DUMMY = NC * MM
NCOLS = FR * 4 * NPF


def _sc_body(trk_hbm, met_hbm, det_hbm, w_hbm, q_hbm,
             xt_hbm, mt_hbm, dt_hbm, hit_hbm, tag_hbm,
             w_v, woff_sc, val_sc, s_sc, idx2_sc,
             q_v, q2d_sc, qoff_sc, t_sc, tc_sc, wt_sc, hit_v,
             trkidx, metidx, detidx, xt_v, mt_v, dt_v, sem):
  cid = lax.axis_index("c")
  sid = lax.axis_index("s")
  wid = sid * NC + cid
  coff = cid * MM

  pltpu.sync_copy(w_hbm.at[pl.ds(sid * WW, WW)], w_v)
  for r in range(WR):
    for l in range(8):
      s = pl.ds(l * 16, 16)
      wv = w_v[pl.ds(r * 128 + l * 16, 16)]
      woff_sc[r, s] = wv + coff
      val_sc[r, s] = lax.iota(jnp.int32, 16) + (sid * WW + r * 128 + l * 16)
  cps = [pltpu.async_copy(val_sc.at[r], tag_hbm.at[woff_sc.at[r]], sem)
         for r in range(WR)]
  for c in cps:
    c.wait()
  plsc.subcore_barrier()

  for _ in range(ROUNDS):
    cps = [pltpu.async_copy(tag_hbm.at[woff_sc.at[r]], s_sc.at[r], sem)
           for r in range(WR)]
    for c in cps:
      c.wait()
    for r in range(WR):
      for l in range(8):
        s = pl.ds(l * 16, 16)
        loser = val_sc[r, s] > s_sc[r, s]
        idx2_sc[r, s] = jnp.where(loser, woff_sc[r, s], DUMMY)
    cps = [pltpu.async_copy(val_sc.at[r], tag_hbm.at[idx2_sc.at[r]], sem)
           for r in range(WR)]
    for c in cps:
      c.wait()
    plsc.subcore_barrier()

  qbase = wid * QW
  pltpu.sync_copy(q_hbm.at[pl.ds(qbase, QW)], q_v)
  for r in range(QR):
    for l in range(8):
      s = pl.ds(l * 16, 16)
      qv = q_v[pl.ds(r * 128 + l * 16, 16)]
      q2d_sc[r, s] = qv
      qoff_sc[r, s] = qv + coff
  cps = [pltpu.async_copy(tag_hbm.at[qoff_sc.at[r]], t_sc.at[r], sem)
         for r in range(QR)]
  for c in cps:
    c.wait()
  for r in range(QR):
    for l in range(8):
      s = pl.ds(l * 16, 16)
      t = t_sc[r, s]
      tc_sc[r, s] = jnp.minimum(jnp.maximum(t, 0), BB - 1)
  cps = [pltpu.async_copy(w_hbm.at[tc_sc.at[r]], wt_sc.at[r], sem)
         for r in range(QR)]
  for c in cps:
    c.wait()
  for r in range(QR):
    for l in range(8):
      s = pl.ds(l * 16, 16)
      hv = jnp.where(wt_sc[r, s] == q2d_sc[r, s], 1, 0)
      hit_v[pl.ds(r * 128 + l * 16, 16)] = hv
      q20 = q2d_sc[r, s] * (FR * 4)
      q5 = q2d_sc[r, s] * FR
      tc5 = tc_sc[r, s] * 5
      for k in range(FR * 4):
        trkidx[k * QR + r, s] = q20 + k
      for k in range(FR):
        metidx[k * QR + r, s] = q5 + k
      for k in range(5):
        detidx[k * QR + r, s] = tc5 + k
  cps = []
  for k in range(FR * 4):
    for r in range(QR):
      cps.append(pltpu.async_copy(
          trk_hbm.at[trkidx.at[k * QR + r]],
          xt_v.at[pl.ds(k * QW + r * 128, 128)], sem))
  for k in range(FR):
    for r in range(QR):
      cps.append(pltpu.async_copy(
          met_hbm.at[metidx.at[k * QR + r]],
          mt_v.at[pl.ds(k * QW + r * 128, 128)], sem))
  for k in range(5):
    for r in range(QR):
      cps.append(pltpu.async_copy(
          det_hbm.at[detidx.at[k * QR + r]],
          dt_v.at[pl.ds(k * QW + r * 128, 128)], sem))
  for c in cps:
    c.wait()
  for k in range(FR * 4):
    pltpu.sync_copy(xt_v.at[pl.ds(k * QW, QW)],
                    xt_hbm.at[pl.ds(k * BB + qbase, QW)])
  for k in range(FR):
    pltpu.sync_copy(mt_v.at[pl.ds(k * QW, QW)],
                    mt_hbm.at[pl.ds(k * BB + qbase, QW)])
  for k in range(5):
    pltpu.sync_copy(dt_v.at[pl.ds(k * QW, QW)],
                    dt_hbm.at[pl.ds(k * BB + qbase, QW)])
  pltpu.sync_copy(hit_v, hit_hbm.at[pl.ds(qbase, QW)])


@functools.lru_cache(maxsize=None)
def _sc_stage():
  mesh = plsc.VectorSubcoreMesh(core_axis_name="c", subcore_axis_name="s",
                                num_cores=NC, num_subcores=NS)
  return pl.kernel(
      _sc_body,
      out_type=(
          jax.ShapeDtypeStruct((FR * 4 * BB,), jnp.float32),
          jax.ShapeDtypeStruct((FR * BB,), jnp.float32),
          jax.ShapeDtypeStruct((5 * BB,), jnp.float32),
          jax.ShapeDtypeStruct((BB,), jnp.int32),
          jax.ShapeDtypeStruct((TAG_LEN,), jnp.int32),
      ),
      mesh=mesh,
      scratch_types=[
          pltpu.VMEM((WW,), jnp.int32),
          pltpu.VMEM((WR, 128), jnp.int32),
          pltpu.VMEM((WR, 128), jnp.int32),
          pltpu.VMEM((WR, 128), jnp.int32),
          pltpu.VMEM((WR, 128), jnp.int32),
          pltpu.VMEM((QW,), jnp.int32),
          pltpu.VMEM((QR, 128), jnp.int32),
          pltpu.VMEM((QR, 128), jnp.int32),
          pltpu.VMEM((QR, 128), jnp.int32),
          pltpu.VMEM((QR, 128), jnp.int32),
          pltpu.VMEM((QR, 128), jnp.int32),
          pltpu.VMEM((QW,), jnp.int32),
          pltpu.VMEM((FR * 4 * QR, 128), jnp.int32),
          pltpu.VMEM((FR * QR, 128), jnp.int32),
          pltpu.VMEM((5 * QR, 128), jnp.int32),
          pltpu.VMEM((FR * 4 * QW,), jnp.float32),
          pltpu.VMEM((FR * QW,), jnp.float32),
          pltpu.VMEM((5 * QW,), jnp.float32),
          pltpu.SemaphoreType.DMA,
      ],
  )


def _tc_body(xt_ref, mt_ref, dt_ref, h_ref, e_ref, t1t_ref, coef_ref,
             shift_ref, o_ref):
  xt = xt_ref[...]
  dt = dt_ref[...]
  ht = h_ref[...] > 0
  dtile = lax.dot_general(t1t_ref[...], dt, (((1,), (0,)), ((), ())),
                          precision=lax.Precision.HIGHEST,
                          preferred_element_type=jnp.float32)
  xsel = jnp.where(ht, dtile, xt)
  xb = lax.dot_general(xsel, e_ref[...], (((0,), (0,)), ((), ())),
                       precision=lax.Precision.HIGHEST,
                       preferred_element_type=jnp.float32)
  phase = xb * coef_ref[...] - shift_ref[...]
  o_ref[:, pl.ds(0, NCOLS)] = jnp.cos(phase)
  moutt = jnp.where(ht, dt[4:5, :], mt_ref[...])
  o_ref[:, pl.ds(NCOLS, FR)] = moutt.T


def _tc_consts():
  dim_t = np.float32(TEMP) ** (
      2.0 * np.floor(np.arange(NPF, dtype=np.float32) / 2.0)
      / np.float32(NPF)).astype(np.float32)
  c = np.arange(NCOLS)
  m32 = c % NPF
  m = np.where(m32 < NPF // 2, m32, m32 - NPF // 2)
  coef = (np.float32(2.0 * np.pi) / dim_t[2 * m]).astype(np.float32)
  shift = np.where(m32 < NPF // 2, np.float32(0.0),
                   np.float32(np.pi / 2)).astype(np.float32)
  e = (c // NPF == np.arange(FR * 4)[:, None]).astype(np.float32)
  t1t = (np.arange(5)[None, :] == np.arange(FR * 4)[:, None] % 4
         ).astype(np.float32)
  return (e, t1t, coef.reshape(1, NCOLS), shift.reshape(1, NCOLS))


def _tc_stage(xt, mt, dt, hit):
  e, t1t, coef, shift = (jnp.asarray(a) for a in _tc_consts())
  bm = 1024
  return pl.pallas_call(
      _tc_body,
      grid=(BB // bm,),
      in_specs=[
          pl.BlockSpec((FR * 4, bm), lambda i: (0, i)),
          pl.BlockSpec((FR, bm), lambda i: (0, i)),
          pl.BlockSpec((5, bm), lambda i: (0, i)),
          pl.BlockSpec((1, bm), lambda i: (0, i)),
          pl.BlockSpec((FR * 4, NCOLS), lambda i: (0, 0)),
          pl.BlockSpec((FR * 4, 5), lambda i: (0, 0)),
          pl.BlockSpec((1, NCOLS), lambda i: (0, 0)),
          pl.BlockSpec((1, NCOLS), lambda i: (0, 0)),
      ],
      out_specs=pl.BlockSpec((bm, NCOLS + FR), lambda i: (i, 0)),
      out_shape=jax.ShapeDtypeStruct((BB, NCOLS + FR), jnp.float32),
  )(xt, mt, dt, hit, e, t1t, coef, shift)


def kernel(tracklets, tracklet_metadata, detections, write_indices,
           query_indices):
  trk = tracklets.reshape(MM * FR * 4)
  met = tracklet_metadata.reshape(MM * FR)
  det = detections.reshape(BB * 5)
  w = write_indices.astype(jnp.int32)
  q = query_indices.astype(jnp.int32)
  xt, mt, dt, hit, _ = _sc_stage()(trk, met, det, w, q)
  return _tc_stage(xt.reshape(FR * 4, BB), mt.reshape(FR, BB),
                   dt.reshape(5, BB), hit.reshape(1, BB))


# speedup vs baseline: 1.1532x; 1.1532x over previous
import functools

import jax
import jax.numpy as jnp
import numpy as np
from jax import lax
from jax.experimental import pallas as pl
from jax.experimental.pallas import tpu as pltpu
from jax.experimental.pallas import tpu_sc as plsc

FR = 5
NPF = 32
TEMP = 10000.0
MM = 1_000_000
BB = 16384
NC, NS = 2, 16
NW = NC * NS
QW = BB // NW
WW = BB // NS
WR = WW // 128
QR = QW // 128
FQR = FR * QR
ROUNDS = 3
TAG_LEN = NC * MM + 128
DUMMY = NC * MM
NCOLS = FR * 4 * NPF


def _sc_body(trk_hbm, met_hbm, det_hbm, w_hbm, q_hbm,
             xt_hbm, mt_hbm, dt_hbm, hit_hbm, tag_hbm,
             w_v, woff_v, val_v, s_v, idx2_v,
             q_v, qoff_v, t_v, tc_v, wt_v, hit_v,
             qf_v, detidx_v, xtv, mtv, dtv, sem):
  cid = lax.axis_index("c")
  sid = lax.axis_index("s")
  wid = sid * NC + cid
  coff = cid * MM

  pltpu.sync_copy(w_hbm.at[pl.ds(sid * WW, WW)], w_v)
  for j in range(WW // 16):
    s = pl.ds(j * 16, 16)
    woff_v[s] = w_v[s] + coff
    val_v[s] = lax.iota(jnp.int32, 16) + (sid * WW + j * 16)
  pltpu.async_copy(val_v, tag_hbm.at[woff_v], sem).wait()
  plsc.subcore_barrier()

  for _ in range(ROUNDS):
    pltpu.async_copy(tag_hbm.at[woff_v], s_v, sem).wait()
    for j in range(WW // 16):
      s = pl.ds(j * 16, 16)
      loser = val_v[s] > s_v[s]
      idx2_v[s] = jnp.where(loser, woff_v[s], DUMMY)
    pltpu.async_copy(val_v, tag_hbm.at[idx2_v], sem).wait()
    plsc.subcore_barrier()

  qbase = wid * QW
  pltpu.sync_copy(q_hbm.at[pl.ds(qbase, QW)], q_v)
  for j in range(QW // 16):
    s = pl.ds(j * 16, 16)
    qoff_v[s] = q_v[s] + coff
  cp_t = pltpu.async_copy(tag_hbm.at[qoff_v], t_v, sem)
  for f in range(FR):
    for j in range(QW // 16):
      s = pl.ds(j * 16, 16)
      qf_v[pl.ds(f * QW + j * 16, 16)] = q_v[s] * FR + f
  cp_x = pltpu.async_copy(trk_hbm.at[qf_v], xtv, sem)
  cp_m = pltpu.async_copy(met_hbm.at[qf_v], mtv, sem)
  cp_t.wait()
  for j in range(QW // 16):
    s = pl.ds(j * 16, 16)
    tc_v[s] = jnp.minimum(jnp.maximum(t_v[s], 0), BB - 1)
  cp_w = pltpu.async_copy(w_hbm.at[tc_v], wt_v, sem)
  for f in range(FR):
    for j in range(QW // 16):
      s = pl.ds(j * 16, 16)
      detidx_v[pl.ds(f * QW + j * 16, 16)] = tc_v[s] * FR + f
  cp_d = pltpu.async_copy(det_hbm.at[detidx_v], dtv, sem)
  cp_w.wait()
  for j in range(QW // 16):
    s = pl.ds(j * 16, 16)
    hit_v[s] = jnp.where(wt_v[s] == q_v[s], 1, 0)
  pltpu.sync_copy(hit_v, hit_hbm.at[pl.ds(qbase, QW)])
  cp_x.wait()
  pltpu.sync_copy(xtv, xt_hbm.at[pl.ds(wid * FR * QW, FR * QW)])
  cp_m.wait()
  pltpu.sync_copy(mtv, mt_hbm.at[pl.ds(wid * FR * QW, FR * QW)])
  cp_d.wait()
  pltpu.sync_copy(dtv, dt_hbm.at[pl.ds(wid * FR * QW, FR * QW)])


@functools.lru_cache(maxsize=None)
def _sc_stage():
  mesh = plsc.VectorSubcoreMesh(core_axis_name="c", subcore_axis_name="s",
                                num_cores=NC, num_subcores=NS)
  return pl.kernel(
      _sc_body,
      out_type=(
          jax.ShapeDtypeStruct((NW * FR * QW, 4), jnp.float32),
          jax.ShapeDtypeStruct((NW * FR * QW,), jnp.float32),
          jax.ShapeDtypeStruct((NW * FR * QW,), jnp.float32),
          jax.ShapeDtypeStruct((BB,), jnp.int32),
          jax.ShapeDtypeStruct((TAG_LEN,), jnp.int32),
      ),
      mesh=mesh,
      compiler_params=pltpu.CompilerParams(use_tc_tiling_on_sc=False),
      scratch_types=[
          pltpu.VMEM((WW,), jnp.int32),
          pltpu.VMEM((WW,), jnp.int32),
          pltpu.VMEM((WW,), jnp.int32),
          pltpu.VMEM((WW,), jnp.int32),
          pltpu.VMEM((WW,), jnp.int32),
          pltpu.VMEM((QW,), jnp.int32),
          pltpu.VMEM((QW,), jnp.int32),
          pltpu.VMEM((QW,), jnp.int32),
          pltpu.VMEM((QW,), jnp.int32),
          pltpu.VMEM((QW,), jnp.int32),
          pltpu.VMEM((QW,), jnp.int32),
          pltpu.VMEM((FR * QW,), jnp.int32),
          pltpu.VMEM((FR * QW,), jnp.int32),
          pltpu.VMEM((FR * QW, 4), jnp.float32),
          pltpu.VMEM((FR * QW,), jnp.float32),
          pltpu.VMEM((FR * QW,), jnp.float32),
          pltpu.SemaphoreType.DMA,
      ],
  )


def _tc_body(xt_ref, mt_ref, dt_ref, h_ref, e_ref, t1t_ref, coef_ref,
             shift_ref, o_ref):
  xt = xt_ref[...]
  dt = dt_ref[...]
  ht = h_ref[...] > 0
  dtile = lax.dot_general(t1t_ref[...], dt, (((1,), (0,)), ((), ())),
                          precision=lax.Precision.HIGHEST,
                          preferred_element_type=jnp.float32)
  xsel = jnp.where(ht, dtile, xt)
  xb = lax.dot_general(xsel, e_ref[...], (((0,), (0,)), ((), ())),
                       precision=lax.Precision.HIGHEST,
                       preferred_element_type=jnp.float32)
  phase = xb * coef_ref[...] - shift_ref[...]
  o_ref[:, pl.ds(0, NCOLS)] = jnp.cos(phase)
  moutt = jnp.where(ht, dt[4:5, :], mt_ref[...])
  o_ref[:, pl.ds(NCOLS, FR)] = moutt.T


def _tc_consts():
  dim_t = np.float32(TEMP) ** (
      2.0 * np.floor(np.arange(NPF, dtype=np.float32) / 2.0)
      / np.float32(NPF)).astype(np.float32)
  c = np.arange(NCOLS)
  m32 = c % NPF
  m = np.where(m32 < NPF // 2, m32, m32 - NPF // 2)
  coef = (np.float32(2.0 * np.pi) / dim_t[2 * m]).astype(np.float32)
  shift = np.where(m32 < NPF // 2, np.float32(0.0),
                   np.float32(np.pi / 2)).astype(np.float32)
  e = (c // NPF == np.arange(FR * 4)[:, None]).astype(np.float32)
  t1t = (np.arange(5)[None, :] == np.arange(FR * 4)[:, None] % 4
         ).astype(np.float32)
  return (e, t1t, coef.reshape(1, NCOLS), shift.reshape(1, NCOLS))


def _tc_stage(xt, mt, dt, hit):
  e, t1t, coef, shift = (jnp.asarray(a) for a in _tc_consts())
  bm = 1024
  return pl.pallas_call(
      _tc_body,
      grid=(BB // bm,),
      in_specs=[
          pl.BlockSpec((FR * 4, bm), lambda i: (0, i)),
          pl.BlockSpec((FR, bm), lambda i: (0, i)),
          pl.BlockSpec((5, bm), lambda i: (0, i)),
          pl.BlockSpec((1, bm), lambda i: (0, i)),
          pl.BlockSpec((FR * 4, NCOLS), lambda i: (0, 0)),
          pl.BlockSpec((FR * 4, 5), lambda i: (0, 0)),
          pl.BlockSpec((1, NCOLS), lambda i: (0, 0)),
          pl.BlockSpec((1, NCOLS), lambda i: (0, 0)),
      ],
      out_specs=pl.BlockSpec((bm, NCOLS + FR), lambda i: (i, 0)),
      out_shape=jax.ShapeDtypeStruct((BB, NCOLS + FR), jnp.float32),
  )(xt, mt, dt, hit, e, t1t, coef, shift)


def kernel(tracklets, tracklet_metadata, detections, write_indices,
           query_indices):
  trk = tracklets.reshape(MM * FR, 4)
  met = tracklet_metadata.reshape(MM * FR)
  det = detections.reshape(BB * 5)
  w = write_indices.astype(jnp.int32)
  q = query_indices.astype(jnp.int32)
  xt4, mt1, dt1, hit, _ = _sc_stage()(trk, met, det, w, q)
  xt = (xt4.reshape(NW, FR, QW, 4)
        .transpose(1, 3, 0, 2).reshape(FR * 4, BB))
  mt = mt1.reshape(NW, FR, QW).transpose(1, 0, 2).reshape(FR, BB)
  dt = dt1.reshape(NW, FR, QW).transpose(1, 0, 2).reshape(FR, BB)
  return _tc_stage(xt, mt, dt, hit.reshape(1, BB))
